# initial kernel scaffold (unmeasured)
import jax
import jax.numpy as jnp
from jax import lax
from jax.experimental import pallas as pl
from jax.experimental.pallas import tpu as pltpu

N_DEV = 4
_GELU_C = 0.7978845608028654


def kernel(x, w_mat):
    m_total, k_per = x.shape
    k_total, n = w_mat.shape
    m_per = m_total // N_DEV

    def body(x_ref, w_ref, out_ref, comm_ref, send_sems, recv_sems):
        me = lax.axis_index("i")

        barrier_sem = pltpu.get_barrier_semaphore()
        for d in range(1, N_DEV):
            peer = lax.rem(me + d, N_DEV)
            pl.semaphore_signal(
                barrier_sem, inc=1,
                device_id=(peer,), device_id_type=pl.DeviceIdType.MESH,
            )
        pl.semaphore_wait(barrier_sem, N_DEV - 1)

        rdmas = []
        for d in range(1, N_DEV):
            target = lax.rem(me + d, N_DEV)
            rdma = pltpu.make_async_remote_copy(
                src_ref=x_ref.at[pl.ds(target * m_per, m_per), :],
                dst_ref=comm_ref.at[d - 1],
                send_sem=send_sems.at[d - 1],
                recv_sem=recv_sems.at[d - 1],
                device_id=(target,),
                device_id_type=pl.DeviceIdType.MESH,
            )
            rdma.start()
            rdmas.append(rdma)

        a_local = x_ref[pl.ds(me * m_per, m_per), :]
        w_local = w_ref[pl.ds(me * k_per, k_per), :]
        acc = jnp.dot(a_local, w_local, preferred_element_type=jnp.float32)

        for d in range(1, N_DEV):
            src = lax.rem(me - d + N_DEV, N_DEV)
            rdmas[d - 1].wait_recv()
            w_blk = w_ref[pl.ds(src * k_per, k_per), :]
            acc = acc + jnp.dot(
                comm_ref[d - 1], w_blk, preferred_element_type=jnp.float32
            )

        for d in range(1, N_DEV):
            rdmas[d - 1].wait_send()

        y = 0.5 * acc * (1.0 + jnp.tanh(_GELU_C * (acc + 0.044715 * acc * acc * acc)))
        out_ref[:, :] = y.astype(jnp.float32)

    return pl.pallas_call(
        body,
        out_shape=jax.ShapeDtypeStruct((m_per, n), jnp.float32),
        in_specs=[
            pl.BlockSpec(memory_space=pltpu.VMEM),
            pl.BlockSpec(memory_space=pltpu.VMEM),
        ],
        out_specs=pl.BlockSpec(memory_space=pltpu.VMEM),
        scratch_shapes=[
            pltpu.VMEM((N_DEV - 1, m_per, k_per), jnp.float32),
            pltpu.SemaphoreType.DMA((N_DEV - 1,)),
            pltpu.SemaphoreType.DMA((N_DEV - 1,)),
        ],
        compiler_params=pltpu.CompilerParams(collective_id=0),
    )(x, w_mat)


# baseline (device time: 115773 ns/iter reference)
import jax
import jax.numpy as jnp
from jax import lax
from jax.experimental import pallas as pl
from jax.experimental.pallas import tpu as pltpu

N_DEV = 4
N_TILE = 512
GELU_ROWS = 256
_GELU_C = 0.7978845608028654


def kernel(x, w_mat):
    m_total, k_per = x.shape
    k_total, n = w_mat.shape
    m_per = m_total // N_DEV

    def body(x_ref, w_ref, out_ref, comm_ref, a_local, w_bufs,
             send_sems, recv_sems, local_sem, w_sems):
        me = lax.axis_index("i")

        barrier_sem = pltpu.get_barrier_semaphore()
        for d in range(1, N_DEV):
            peer = lax.rem(me + d, N_DEV)
            pl.semaphore_signal(
                barrier_sem, inc=1,
                device_id=(peer,), device_id_type=pl.DeviceIdType.MESH,
            )
        pl.semaphore_wait(barrier_sem, N_DEV - 1)

        rdmas = []
        for d in range(1, N_DEV):
            target = lax.rem(me + d, N_DEV)
            rdma = pltpu.make_async_remote_copy(
                src_ref=x_ref.at[pl.ds(target * m_per, m_per), :],
                dst_ref=comm_ref.at[d - 1],
                send_sem=send_sems.at[d - 1],
                recv_sem=recv_sems.at[d - 1],
                device_id=(target,),
                device_id_type=pl.DeviceIdType.MESH,
            )
            rdma.start()
            rdmas.append(rdma)

        local_cp = pltpu.make_async_copy(
            x_ref.at[pl.ds(me * m_per, m_per), :], a_local, local_sem
        )
        local_cp.start()

        def w_src(step):
            return lax.rem(me - step + N_DEV, N_DEV)

        def start_w(step):
            slot = step % 2
            cp = pltpu.make_async_copy(
                w_ref.at[pl.ds(w_src(step) * k_per, k_per), :],
                w_bufs.at[slot],
                w_sems.at[slot],
            )
            cp.start()
            return cp

        w_cps = [start_w(0), start_w(1)]

        for step in range(N_DEV):
            slot = step % 2
            if step == 0:
                local_cp.wait()
            else:
                rdmas[step - 1].wait_recv()
            w_cps[step].wait()
            a_blk = a_local if step == 0 else comm_ref.at[step - 1]
            for nt in range(n // N_TILE):
                ncols = pl.ds(nt * N_TILE, N_TILE)
                partial = jnp.dot(
                    a_blk[:, :], w_bufs[slot, :, ncols],
                    preferred_element_type=jnp.float32,
                )
                if step == 0:
                    out_ref[:, ncols] = partial
                else:
                    out_ref[:, ncols] = out_ref[:, ncols] + partial
            if step + 2 < N_DEV:
                w_cps.append(start_w(step + 2))

        for r in range(0, m_per, GELU_ROWS):
            rows = pl.ds(r, GELU_ROWS)
            v = out_ref[rows, :]
            out_ref[rows, :] = 0.5 * v * (
                1.0 + jnp.tanh(_GELU_C * (v + 0.044715 * v * v * v))
            )

        for d in range(1, N_DEV):
            rdmas[d - 1].wait_send()

    return pl.pallas_call(
        body,
        out_shape=jax.ShapeDtypeStruct((m_per, n), jnp.float32),
        in_specs=[
            pl.BlockSpec(memory_space=pltpu.MemorySpace.HBM),
            pl.BlockSpec(memory_space=pltpu.MemorySpace.HBM),
        ],
        out_specs=pl.BlockSpec(memory_space=pltpu.VMEM),
        scratch_shapes=[
            pltpu.VMEM((N_DEV - 1, m_per, k_per), jnp.float32),
            pltpu.VMEM((m_per, k_per), jnp.float32),
            pltpu.VMEM((2, k_per, n), jnp.float32),
            pltpu.SemaphoreType.DMA((N_DEV - 1,)),
            pltpu.SemaphoreType.DMA((N_DEV - 1,)),
            pltpu.SemaphoreType.DMA,
            pltpu.SemaphoreType.DMA((2,)),
        ],
        compiler_params=pltpu.CompilerParams(
            collective_id=0,
            vmem_limit_bytes=60 * 1024 * 1024,
        ),
    )(x, w_mat)
